# baseline (speedup 1.0000x reference)
import functools

import jax
import jax.numpy as jnp
from jax import lax
from jax.experimental import pallas as pl
from jax.experimental.pallas import tpu as pltpu
from jax.experimental.pallas import tpu_sc as plsc

_R = 4
_NBUF = 2


def kernel(x, table):
    b, l = x.shape
    _, d = table.shape
    info = plsc.get_sparse_core_info()
    nw = info.num_cores * info.num_subcores
    rows_per_w = b // nw
    steps = rows_per_w // _R
    mesh = plsc.VectorSubcoreMesh(core_axis_name="c", subcore_axis_name="s")

    @functools.partial(
        pl.kernel,
        out_type=jax.ShapeDtypeStruct((b, l, d), jnp.float32),
        mesh=mesh,
        scratch_types=[
            pltpu.VMEM((_NBUF, _R, l), jnp.int32),
            pltpu.VMEM((_NBUF, _R, l, d), jnp.float32),
            pltpu.SemaphoreType.DMA((_NBUF,)),
            pltpu.SemaphoreType.DMA((_NBUF,)),
            pltpu.SemaphoreType.DMA((_NBUF,)),
        ],
        compiler_params=pltpu.CompilerParams(use_tc_tiling_on_sc=False),
    )
    def emb(x_hbm, table_hbm, out_hbm, idx_v, rows_v, idx_sem, gat_sem, out_sem):
        wid = lax.axis_index("s") * info.num_cores + lax.axis_index("c")
        base = wid * rows_per_w

        for bb in range(_NBUF):
            pltpu.async_copy(
                x_hbm.at[pl.ds(base + bb * _R, _R)], idx_v.at[bb],
                idx_sem.at[bb])

        @pl.loop(0, steps)
        def _step(i):
            bb = lax.rem(i, _NBUF)
            off = base + i * _R

            @pl.when(i >= _NBUF)
            def _():
                pltpu.make_async_copy(
                    rows_v.at[bb], out_hbm.at[pl.ds(0, _R)],
                    out_sem.at[bb]).wait()

            pltpu.make_async_copy(
                x_hbm.at[pl.ds(off, _R)], idx_v.at[bb],
                idx_sem.at[bb]).wait()

            descs = [
                pltpu.async_copy(
                    table_hbm.at[idx_v.at[bb, r]], rows_v.at[bb, r],
                    gat_sem.at[bb])
                for r in range(_R)
            ]
            for desc in descs:
                desc.wait()

            pltpu.async_copy(
                rows_v.at[bb], out_hbm.at[pl.ds(off, _R)], out_sem.at[bb])

            @pl.when(i + _NBUF < steps)
            def _():
                pltpu.async_copy(
                    x_hbm.at[pl.ds(off + _NBUF * _R, _R)],
                    idx_v.at[bb], idx_sem.at[bb])

        for bb in range(_NBUF):
            pltpu.make_async_copy(
                rows_v.at[bb], out_hbm.at[pl.ds(0, _R)],
                out_sem.at[bb]).wait()

    return emb(x, table)


# speedup vs baseline: 1.2943x; 1.2943x over previous
import functools

import jax
import jax.numpy as jnp
from jax import lax
from jax.experimental import pallas as pl
from jax.experimental.pallas import tpu as pltpu
from jax.experimental.pallas import tpu_sc as plsc

_NBUF = 4


def kernel(x, table):
    b, l = x.shape
    v, d = table.shape
    n = b * l
    dp = 128
    table_p = jnp.pad(table, ((0, 0), (0, dp - d)))
    x_flat = x.reshape(n)
    info = plsc.get_sparse_core_info()
    nw = info.num_cores * info.num_subcores
    rows_per_w = b // nw
    groups = rows_per_w // _NBUF
    mesh = plsc.VectorSubcoreMesh(core_axis_name="c", subcore_axis_name="s")

    @functools.partial(
        pl.kernel,
        out_type=jax.ShapeDtypeStruct((b, l, dp), jnp.float32),
        mesh=mesh,
        scratch_types=(
            [pltpu.VMEM((l,), jnp.int32) for _ in range(_NBUF)]
            + [pltpu.VMEM((l, dp), jnp.float32) for _ in range(_NBUF)]
            + [pltpu.SemaphoreType.DMA((_NBUF,)),
               pltpu.SemaphoreType.DMA((_NBUF,)),
               pltpu.SemaphoreType.DMA((_NBUF,))]
        ),
        compiler_params=pltpu.CompilerParams(use_tc_tiling_on_sc=True),
    )
    def emb(x_hbm, table_hbm, out_hbm, *refs):
        idx_v = refs[:_NBUF]
        rows_v = refs[_NBUF:2 * _NBUF]
        idx_sem, gat_sem, out_sem = refs[2 * _NBUF:]
        wid = lax.axis_index("s") * info.num_cores + lax.axis_index("c")
        base = wid * rows_per_w

        for bb in range(_NBUF):
            pltpu.async_copy(
                x_hbm.at[pl.ds((base + bb) * l, l)], idx_v[bb],
                idx_sem.at[bb])

        @pl.loop(0, groups)
        def _grp(g):
            s0 = base + g * _NBUF
            descs = []
            for bb in range(_NBUF):
                @pl.when(g > 0)
                def _():
                    pltpu.make_async_copy(
                        rows_v[bb], out_hbm.at[0], out_sem.at[bb]).wait()

                pltpu.make_async_copy(
                    x_hbm.at[pl.ds(0, l)], idx_v[bb], idx_sem.at[bb]).wait()

                descs.append(pltpu.async_copy(
                    table_hbm.at[idx_v[bb]], rows_v[bb], gat_sem.at[bb]))

            for bb in range(_NBUF):
                descs[bb].wait()
                pltpu.async_copy(rows_v[bb], out_hbm.at[s0 + bb],
                                 out_sem.at[bb])

                @pl.when(g + 1 < groups)
                def _():
                    pltpu.async_copy(
                        x_hbm.at[pl.ds((s0 + bb + _NBUF) * l, l)],
                        idx_v[bb], idx_sem.at[bb])

        for bb in range(_NBUF):
            pltpu.make_async_copy(
                rows_v[bb], out_hbm.at[0], out_sem.at[bb]).wait()

    out_full = emb(x_flat, table_p)
    return out_full[:, :, :d]
